# baseline (speedup 1.0000x reference)
import functools

import jax
import jax.numpy as jnp
from jax import lax
from jax.experimental import pallas as pl
from jax.experimental.pallas import tpu as pltpu
from jax.experimental.pallas import tpu_sc as plsc

N_ROWS = 65536
N_COLS = 512
LANES = 16
NC = 2
NS = 16
NW = NC * NS
ROWS_PER_W = N_ROWS // NW
CHUNK_R = 32
N_CHUNKS = ROWS_PER_W // CHUNK_R
BLKS = N_COLS // LANES
GRP = 8
N_GRPS = BLKS // GRP
IN_DEPTH = 4
OUT_DEPTH = 2


@functools.partial(
    pl.kernel,
    out_type=jax.ShapeDtypeStruct((N_ROWS, N_COLS), jnp.float32),
    mesh=plsc.VectorSubcoreMesh(core_axis_name="c", subcore_axis_name="s"),
    compiler_params=pltpu.CompilerParams(needs_layout_passes=False),
    scratch_types=(
        [pltpu.VMEM((N_COLS,), jnp.int32)]
        + [pltpu.VMEM((CHUNK_R, N_COLS), jnp.float32)] * (IN_DEPTH + OUT_DEPTH)
        + [pltpu.SemaphoreType.DMA] * (IN_DEPTH + OUT_DEPTH)
    ),
)
def _permute_sc(x_hbm, perm_hbm, out_hbm, perm_v, *bufs_and_sems):
    in_bufs = bufs_and_sems[:IN_DEPTH]
    out_bufs = bufs_and_sems[IN_DEPTH:IN_DEPTH + OUT_DEPTH]
    in_sems = bufs_and_sems[IN_DEPTH + OUT_DEPTH:2 * IN_DEPTH + OUT_DEPTH]
    out_sems = bufs_and_sems[2 * IN_DEPTH + OUT_DEPTH:]

    wid = lax.axis_index("s") * NC + lax.axis_index("c")
    pltpu.sync_copy(perm_hbm, perm_v)
    row0 = wid * ROWS_PER_W

    def start_in(chunk, slot):
        pltpu.make_async_copy(
            x_hbm.at[pl.ds(row0 + chunk * CHUNK_R, CHUNK_R), :],
            in_bufs[slot], in_sems[slot]).start()

    for c in range(IN_DEPTH - 1):
        start_in(c, c)

    def permute_chunk(in_v, out_v):
        for g in range(N_GRPS):
            idxs = [perm_v[pl.ds(LANES * (g * GRP + k), LANES)]
                    for k in range(GRP)]

            @plsc.parallel_loop(0, CHUNK_R, unroll=4)
            def _(r):
                row_idx = jnp.full((LANES,), r, dtype=jnp.int32)
                for k in range(GRP):
                    out_v[r, pl.ds(LANES * (g * GRP + k), LANES)] = (
                        plsc.load_gather(in_v, [row_idx, idxs[k]]))

    def round_body(i, carry):
        for pi in range(IN_DEPTH):
            chunk = IN_DEPTH * i + pi
            po = pi % OUT_DEPTH
            in_v, sem_i = in_bufs[pi], in_sems[pi]
            out_v, sem_o = out_bufs[po], out_sems[po]

            pltpu.make_async_copy(
                x_hbm.at[pl.ds(0, CHUNK_R), :], in_v, sem_i).wait()

            @pl.when(chunk + IN_DEPTH - 1 < N_CHUNKS)
            def _():
                start_in(chunk + IN_DEPTH - 1, (pi + IN_DEPTH - 1) % IN_DEPTH)

            @pl.when(chunk >= OUT_DEPTH)
            def _():
                pltpu.make_async_copy(
                    out_v, out_hbm.at[pl.ds(0, CHUNK_R), :], sem_o).wait()

            permute_chunk(in_v, out_v)
            pltpu.make_async_copy(
                out_v, out_hbm.at[pl.ds(row0 + chunk * CHUNK_R, CHUNK_R), :],
                sem_o).start()
        return carry

    lax.fori_loop(0, N_CHUNKS // IN_DEPTH, round_body, 0)

    for po in range(OUT_DEPTH):
        pltpu.make_async_copy(
            out_bufs[po], out_hbm.at[pl.ds(0, CHUNK_R), :], out_sems[po]).wait()


def kernel(x, perm):
    return _permute_sc(x, perm)


# speedup vs baseline: 1.0254x; 1.0254x over previous
import functools

import jax
import jax.numpy as jnp
from jax import lax
from jax.experimental import pallas as pl
from jax.experimental.pallas import tpu as pltpu
from jax.experimental.pallas import tpu_sc as plsc

N_ROWS = 65536
N_COLS = 512
LANES = 16
NC = 2
NS = 16
NW = NC * NS
ROWS_PER_W = N_ROWS // NW
CHUNK_R = 32
N_CHUNKS = ROWS_PER_W // CHUNK_R
BLKS = N_COLS // LANES
GRP = 16
N_GRPS = BLKS // GRP
IN_DEPTH = 4
OUT_DEPTH = 2


@functools.partial(
    pl.kernel,
    out_type=jax.ShapeDtypeStruct((N_ROWS, N_COLS), jnp.float32),
    mesh=plsc.VectorSubcoreMesh(core_axis_name="c", subcore_axis_name="s"),
    compiler_params=pltpu.CompilerParams(needs_layout_passes=False),
    scratch_types=(
        [pltpu.VMEM((N_COLS,), jnp.int32)]
        + [pltpu.VMEM((CHUNK_R, N_COLS), jnp.float32)] * (IN_DEPTH + OUT_DEPTH)
        + [pltpu.SemaphoreType.DMA] * (IN_DEPTH + OUT_DEPTH)
    ),
)
def _permute_sc(x_hbm, perm_hbm, out_hbm, perm_v, *bufs_and_sems):
    in_bufs = bufs_and_sems[:IN_DEPTH]
    out_bufs = bufs_and_sems[IN_DEPTH:IN_DEPTH + OUT_DEPTH]
    in_sems = bufs_and_sems[IN_DEPTH + OUT_DEPTH:2 * IN_DEPTH + OUT_DEPTH]
    out_sems = bufs_and_sems[2 * IN_DEPTH + OUT_DEPTH:]

    wid = lax.axis_index("s") * NC + lax.axis_index("c")
    pltpu.sync_copy(perm_hbm, perm_v)
    row0 = wid * ROWS_PER_W

    def start_in(chunk, slot):
        pltpu.make_async_copy(
            x_hbm.at[pl.ds(row0 + chunk * CHUNK_R, CHUNK_R), :],
            in_bufs[slot], in_sems[slot]).start()

    for c in range(IN_DEPTH - 1):
        start_in(c, c)

    def permute_chunk(in_v, out_v):
        for g in range(N_GRPS):
            idxs = [perm_v[pl.ds(LANES * (g * GRP + k), LANES)]
                    for k in range(GRP)]

            @plsc.parallel_loop(0, CHUNK_R, unroll=2)
            def _(r):
                row_idx = jnp.full((LANES,), r, dtype=jnp.int32)
                for k in range(GRP):
                    out_v[r, pl.ds(LANES * (g * GRP + k), LANES)] = (
                        plsc.load_gather(in_v, [row_idx, idxs[k]]))

    def round_body(i, carry):
        for pi in range(IN_DEPTH):
            chunk = IN_DEPTH * i + pi
            po = pi % OUT_DEPTH
            in_v, sem_i = in_bufs[pi], in_sems[pi]
            out_v, sem_o = out_bufs[po], out_sems[po]

            pltpu.make_async_copy(
                x_hbm.at[pl.ds(0, CHUNK_R), :], in_v, sem_i).wait()

            @pl.when(chunk + IN_DEPTH - 1 < N_CHUNKS)
            def _():
                start_in(chunk + IN_DEPTH - 1, (pi + IN_DEPTH - 1) % IN_DEPTH)

            @pl.when(chunk >= OUT_DEPTH)
            def _():
                pltpu.make_async_copy(
                    out_v, out_hbm.at[pl.ds(0, CHUNK_R), :], sem_o).wait()

            permute_chunk(in_v, out_v)
            pltpu.make_async_copy(
                out_v, out_hbm.at[pl.ds(row0 + chunk * CHUNK_R, CHUNK_R), :],
                sem_o).start()
        return carry

    lax.fori_loop(0, N_CHUNKS // IN_DEPTH, round_body, 0)

    for po in range(OUT_DEPTH):
        pltpu.make_async_copy(
            out_bufs[po], out_hbm.at[pl.ds(0, CHUNK_R), :], out_sems[po]).wait()


def kernel(x, perm):
    return _permute_sc(x, perm)


# speedup vs baseline: 1.0314x; 1.0059x over previous
import functools

import jax
import jax.numpy as jnp
from jax import lax
from jax.experimental import pallas as pl
from jax.experimental.pallas import tpu as pltpu
from jax.experimental.pallas import tpu_sc as plsc

N_ROWS = 65536
N_COLS = 512
LANES = 16
NC = 2
NS = 16
NW = NC * NS
ROWS_PER_W = N_ROWS // NW
CHUNK_R = 32
N_CHUNKS = ROWS_PER_W // CHUNK_R
BLKS = N_COLS // LANES
GRP = 32
N_GRPS = BLKS // GRP
IN_DEPTH = 4
OUT_DEPTH = 2


@functools.partial(
    pl.kernel,
    out_type=jax.ShapeDtypeStruct((N_ROWS, N_COLS), jnp.float32),
    mesh=plsc.VectorSubcoreMesh(core_axis_name="c", subcore_axis_name="s"),
    compiler_params=pltpu.CompilerParams(needs_layout_passes=False),
    scratch_types=(
        [pltpu.VMEM((N_COLS,), jnp.int32)]
        + [pltpu.VMEM((CHUNK_R, N_COLS), jnp.float32)] * (IN_DEPTH + OUT_DEPTH)
        + [pltpu.SemaphoreType.DMA] * (IN_DEPTH + OUT_DEPTH)
    ),
)
def _permute_sc(x_hbm, perm_hbm, out_hbm, perm_v, *bufs_and_sems):
    in_bufs = bufs_and_sems[:IN_DEPTH]
    out_bufs = bufs_and_sems[IN_DEPTH:IN_DEPTH + OUT_DEPTH]
    in_sems = bufs_and_sems[IN_DEPTH + OUT_DEPTH:2 * IN_DEPTH + OUT_DEPTH]
    out_sems = bufs_and_sems[2 * IN_DEPTH + OUT_DEPTH:]

    wid = lax.axis_index("s") * NC + lax.axis_index("c")
    pltpu.sync_copy(perm_hbm, perm_v)
    row0 = wid * ROWS_PER_W

    def start_in(chunk, slot):
        pltpu.make_async_copy(
            x_hbm.at[pl.ds(row0 + chunk * CHUNK_R, CHUNK_R), :],
            in_bufs[slot], in_sems[slot]).start()

    for c in range(IN_DEPTH - 1):
        start_in(c, c)

    def permute_chunk(in_v, out_v):
        for g in range(N_GRPS):
            idxs = [perm_v[pl.ds(LANES * (g * GRP + k), LANES)]
                    for k in range(GRP)]

            @plsc.parallel_loop(0, CHUNK_R, unroll=1)
            def _(r):
                row_idx = jnp.full((LANES,), r, dtype=jnp.int32)
                for k in range(GRP):
                    out_v[r, pl.ds(LANES * (g * GRP + k), LANES)] = (
                        plsc.load_gather(in_v, [row_idx, idxs[k]]))

    def round_body(i, carry):
        for pi in range(IN_DEPTH):
            chunk = IN_DEPTH * i + pi
            po = pi % OUT_DEPTH
            in_v, sem_i = in_bufs[pi], in_sems[pi]
            out_v, sem_o = out_bufs[po], out_sems[po]

            pltpu.make_async_copy(
                x_hbm.at[pl.ds(0, CHUNK_R), :], in_v, sem_i).wait()

            @pl.when(chunk + IN_DEPTH - 1 < N_CHUNKS)
            def _():
                start_in(chunk + IN_DEPTH - 1, (pi + IN_DEPTH - 1) % IN_DEPTH)

            @pl.when(chunk >= OUT_DEPTH)
            def _():
                pltpu.make_async_copy(
                    out_v, out_hbm.at[pl.ds(0, CHUNK_R), :], sem_o).wait()

            permute_chunk(in_v, out_v)
            pltpu.make_async_copy(
                out_v, out_hbm.at[pl.ds(row0 + chunk * CHUNK_R, CHUNK_R), :],
                sem_o).start()
        return carry

    lax.fori_loop(0, N_CHUNKS // IN_DEPTH, round_body, 0)

    for po in range(OUT_DEPTH):
        pltpu.make_async_copy(
            out_bufs[po], out_hbm.at[pl.ds(0, CHUNK_R), :], out_sems[po]).wait()


def kernel(x, perm):
    return _permute_sc(x, perm)


# speedup vs baseline: 1.0322x; 1.0008x over previous
import functools

import jax
import jax.numpy as jnp
from jax import lax
from jax.experimental import pallas as pl
from jax.experimental.pallas import tpu as pltpu
from jax.experimental.pallas import tpu_sc as plsc

N_ROWS = 65536
N_COLS = 512
LANES = 16
NC = 2
NS = 16
NW = NC * NS
ROWS_PER_W = N_ROWS // NW
CHUNK_R = 32
N_CHUNKS = ROWS_PER_W // CHUNK_R
BLKS = N_COLS // LANES
GRP = 32
N_GRPS = BLKS // GRP
IN_DEPTH = 4
OUT_DEPTH = 2


@functools.partial(
    pl.kernel,
    out_type=jax.ShapeDtypeStruct((N_ROWS, N_COLS), jnp.float32),
    mesh=plsc.VectorSubcoreMesh(core_axis_name="c", subcore_axis_name="s"),
    compiler_params=pltpu.CompilerParams(
        needs_layout_passes=False,
        disable_bounds_checks=True,
        disable_semaphore_checks=True,
    ),
    scratch_types=(
        [pltpu.VMEM((N_COLS,), jnp.int32)]
        + [pltpu.VMEM((CHUNK_R, N_COLS), jnp.float32)] * (IN_DEPTH + OUT_DEPTH)
        + [pltpu.SemaphoreType.DMA] * (IN_DEPTH + OUT_DEPTH)
    ),
)
def _permute_sc(x_hbm, perm_hbm, out_hbm, perm_v, *bufs_and_sems):
    in_bufs = bufs_and_sems[:IN_DEPTH]
    out_bufs = bufs_and_sems[IN_DEPTH:IN_DEPTH + OUT_DEPTH]
    in_sems = bufs_and_sems[IN_DEPTH + OUT_DEPTH:2 * IN_DEPTH + OUT_DEPTH]
    out_sems = bufs_and_sems[2 * IN_DEPTH + OUT_DEPTH:]

    wid = lax.axis_index("s") * NC + lax.axis_index("c")
    pltpu.sync_copy(perm_hbm, perm_v)
    row0 = wid * ROWS_PER_W

    def start_in(chunk, slot):
        pltpu.make_async_copy(
            x_hbm.at[pl.ds(row0 + chunk * CHUNK_R, CHUNK_R), :],
            in_bufs[slot], in_sems[slot]).start()

    for c in range(IN_DEPTH - 1):
        start_in(c, c)

    def permute_chunk(in_v, out_v):
        for g in range(N_GRPS):
            idxs = [perm_v[pl.ds(LANES * (g * GRP + k), LANES)]
                    for k in range(GRP)]

            @plsc.parallel_loop(0, CHUNK_R, unroll=1)
            def _(r):
                row_idx = jnp.full((LANES,), r, dtype=jnp.int32)
                for k in range(GRP):
                    out_v[r, pl.ds(LANES * (g * GRP + k), LANES)] = (
                        plsc.load_gather(in_v, [row_idx, idxs[k]]))

    def round_body(i, carry):
        for pi in range(IN_DEPTH):
            chunk = IN_DEPTH * i + pi
            po = pi % OUT_DEPTH
            in_v, sem_i = in_bufs[pi], in_sems[pi]
            out_v, sem_o = out_bufs[po], out_sems[po]

            pltpu.make_async_copy(
                x_hbm.at[pl.ds(0, CHUNK_R), :], in_v, sem_i).wait()

            @pl.when(chunk + IN_DEPTH - 1 < N_CHUNKS)
            def _():
                start_in(chunk + IN_DEPTH - 1, (pi + IN_DEPTH - 1) % IN_DEPTH)

            @pl.when(chunk >= OUT_DEPTH)
            def _():
                pltpu.make_async_copy(
                    out_v, out_hbm.at[pl.ds(0, CHUNK_R), :], sem_o).wait()

            permute_chunk(in_v, out_v)
            pltpu.make_async_copy(
                out_v, out_hbm.at[pl.ds(row0 + chunk * CHUNK_R, CHUNK_R), :],
                sem_o).start()
        return carry

    lax.fori_loop(0, N_CHUNKS // IN_DEPTH, round_body, 0)

    for po in range(OUT_DEPTH):
        pltpu.make_async_copy(
            out_bufs[po], out_hbm.at[pl.ds(0, CHUNK_R), :], out_sems[po]).wait()


def kernel(x, perm):
    return _permute_sc(x, perm)
